# baseline (speedup 1.0000x reference)
import functools

import jax
import jax.numpy as jnp
from jax import lax
from jax.experimental import pallas as pl
from jax.experimental.pallas import tpu as pltpu
from jax.experimental.pallas import tpu_sc as plsc

N = 10000
NP = 10240
E = 320000
EPAD = 327680
H = 128
VOCAB = 1001

ROWS_PER_TILE_A = NP // 32
ROWS_PER_TILE_B = NP // 16
EDGES_PER_TILE = EPAD // 32
EDGE_CHUNKS = EDGES_PER_TILE // 128

_mesh = plsc.VectorSubcoreMesh(core_axis_name="c", subcore_axis_name="s")


@functools.partial(
    pl.kernel,
    mesh=_mesh,
    out_type=jax.ShapeDtypeStruct((NP, H), jnp.float32),
    scratch_types=[
        pltpu.VMEM((ROWS_PER_TILE_A,), jnp.int32),
        pltpu.VMEM((ROWS_PER_TILE_A,), jnp.int32),
        pltpu.VMEM((64, H), jnp.float32),
        pltpu.VMEM((64, H), jnp.float32),
        pltpu.SemaphoreType.DMA,
    ],
)
def _sc_embed(f0_hbm, f1_hbm, kt_hbm, vt_hbm, h_hbm, fi0, fi1, kbuf, vbuf, sem):
    c = lax.axis_index("c")
    s = lax.axis_index("s")
    wid = s * 2 + c
    n0 = wid * ROWS_PER_TILE_A
    pltpu.sync_copy(f0_hbm.at[pl.ds(n0, ROWS_PER_TILE_A)], fi0)
    pltpu.sync_copy(f1_hbm.at[pl.ds(n0, ROWS_PER_TILE_A)], fi1)
    for i in range(ROWS_PER_TILE_A // 64):
        pltpu.async_copy(kt_hbm.at[fi0.at[pl.ds(64 * i, 64)]], kbuf, sem).wait()
        pltpu.async_copy(vt_hbm.at[fi1.at[pl.ds(64 * i, 64)]], vbuf, sem).wait()

        def relu_body(r, carry):
            for j in range(H // 16):
                kbuf[r, pl.ds(16 * j, 16)] = jnp.maximum(
                    kbuf[r, pl.ds(16 * j, 16)] + vbuf[r, pl.ds(16 * j, 16)],
                    0.0)
            return carry
        lax.fori_loop(0, 64, relu_body, 0)
        pltpu.sync_copy(kbuf, h_hbm.at[pl.ds(n0 + 64 * i, 64)])


@functools.partial(
    pl.kernel,
    mesh=_mesh,
    out_type=jax.ShapeDtypeStruct((2, NP, H), jnp.float32),
    scratch_types=[
        pltpu.VMEM((4, 128), jnp.int32),
        pltpu.VMEM((4, 128), jnp.int32),
        pltpu.VMEM((128, H), jnp.float32),
        pltpu.VMEM((128, H), jnp.float32),
        pltpu.VMEM_SHARED((NP, H), jnp.float32),
        pltpu.SemaphoreType.DMA,
        pltpu.SemaphoreType.DMA,
        pltpu.SemaphoreType.DMA,
    ],
)
def _sc_edges(h_hbm, src_hbm, dst_hbm, out_hbm,
              isrc, idst, eb0, eb1, acc_sh, isem, gsem, ssem):
    c = lax.axis_index("c")
    s = lax.axis_index("s")
    r0 = s * ROWS_PER_TILE_B
    e0 = (c * 16 + s) * EDGES_PER_TILE
    eb = [eb0, eb1]

    @pl.when(c == 0)
    def _():
        pltpu.sync_copy(h_hbm.at[pl.ds(r0, ROWS_PER_TILE_B)],
                        acc_sh.at[pl.ds(r0, ROWS_PER_TILE_B)])

    @pl.when(c == 1)
    def _():
        def zero_body(r, carry):
            for j in range(H // 16):
                eb0[r, pl.ds(16 * j, 16)] = jnp.zeros((16,), jnp.float32)
            return carry
        lax.fori_loop(0, 128, zero_body, 0)
        for i in range(ROWS_PER_TILE_B // 128):
            pltpu.sync_copy(eb0, acc_sh.at[pl.ds(r0 + 128 * i, 128)])

    plsc.subcore_barrier()

    def fire_idx(j, r):
        pltpu.async_copy(src_hbm.at[pl.ds(e0 + j * 128, 128)], isrc.at[r], isem)
        pltpu.async_copy(dst_hbm.at[pl.ds(e0 + j * 128, 128)], idst.at[r], isem)

    def wait_idx(r):
        pltpu.make_async_copy(src_hbm.at[pl.ds(0, 128)], isrc.at[r], isem).wait()
        pltpu.make_async_copy(dst_hbm.at[pl.ds(0, 128)], idst.at[r], isem).wait()

    def fire_g(r, buf):
        pltpu.async_copy(h_hbm.at[isrc.at[r]], buf, gsem)

    def wait_g(buf):
        pltpu.make_async_copy(h_hbm.at[isrc.at[0]], buf, gsem).wait()

    def fire_s(r, buf):
        pltpu.async_copy(buf, acc_sh.at[idst.at[r]], ssem, add=True)

    def wait_s(buf):
        pltpu.make_async_copy(buf, acc_sh.at[idst.at[0]], ssem).wait()

    def step(j, k, fire_idx_f=True, wait_idx_f=True, wait_s_f=True,
             fire_g_f=True):
        if fire_idx_f:
            fire_idx(j + 2, (k + 2) % 4)
        if wait_idx_f:
            wait_idx((k + 1) % 4)
        if wait_s_f:
            wait_s(eb[(k + 1) % 2])
        if fire_g_f:
            fire_g((k + 1) % 4, eb[(k + 1) % 2])
        wait_g(eb[k % 2])
        fire_s(k, eb[k % 2])

    fire_idx(0, 0)
    fire_idx(1, 1)
    wait_idx(0)
    fire_g(0, eb0)
    step(0, 0, wait_s_f=False)
    step(1, 1)

    def body(t, carry):
        j = 4 * t + 2
        for k in range(4):
            step(j + k, (2 + k) % 4)
        return carry
    lax.fori_loop(0, (EDGE_CHUNKS - 8) // 4, body, 0)

    for j in range(EDGE_CHUNKS - 6, EDGE_CHUNKS):
        last = j == EDGE_CHUNKS - 1
        step(j, j % 4, fire_idx_f=j + 2 < EDGE_CHUNKS,
             wait_idx_f=not last, fire_g_f=not last)
    wait_s(eb[(EDGE_CHUNKS - 1) % 2])

    plsc.subcore_barrier()

    for i in range(ROWS_PER_TILE_B // 128):
        pltpu.sync_copy(acc_sh.at[pl.ds(r0 + 128 * i, 128)], eb0)
        pltpu.sync_copy(eb0, out_hbm.at[c].at[pl.ds(r0 + 128 * i, 128)])


BLK = 1024


def _mlp_body(ha_ref, hb_ref, w1_ref, b1_ref, w2_ref, b2_ref, wc_ref, o_ref):
    h = ha_ref[0] + hb_ref[0]
    z = jnp.dot(h, w1_ref[...], preferred_element_type=jnp.float32)
    z = jnp.maximum(z + b1_ref[...], 0.0)
    z = jnp.dot(z, w2_ref[...], preferred_element_type=jnp.float32) + b2_ref[...]
    o_ref[...] = jnp.dot(z, wc_ref[...], preferred_element_type=jnp.float32)


def _mlp(hs, W1, b1, W2, b2, Wc):
    return pl.pallas_call(
        _mlp_body,
        grid=(NP // BLK,),
        in_specs=[
            pl.BlockSpec((1, BLK, H), lambda i: (0, i, 0)),
            pl.BlockSpec((1, BLK, H), lambda i: (1, i, 0)),
            pl.BlockSpec((H, H), lambda i: (0, 0)),
            pl.BlockSpec((1, H), lambda i: (0, 0)),
            pl.BlockSpec((H, H), lambda i: (0, 0)),
            pl.BlockSpec((1, H), lambda i: (0, 0)),
            pl.BlockSpec((H, H), lambda i: (0, 0)),
        ],
        out_specs=pl.BlockSpec((BLK, H), lambda i: (i, 0)),
        out_shape=jax.ShapeDtypeStruct((NP, H), jnp.float32),
    )(hs, hs, W1, b1.reshape(1, H), W2, b2.reshape(1, H), Wc)


def kernel(feats, edge_index, key_table, val_table, W1, b1, W2, b2, Wc):
    f0 = jnp.pad(feats[:, 0], (0, NP - N))
    f1 = jnp.pad(feats[:, 1], (0, NP - N))
    srcp = jnp.full((EPAD,), NP - 1, jnp.int32).at[:E].set(edge_index[0])
    dstp = jnp.full((EPAD,), NP - 1, jnp.int32).at[:E].set(edge_index[1])
    h = _sc_embed(f0, f1, key_table, val_table)
    hs = _sc_edges(h, srcp, dstp)
    out = _mlp(hs, W1, b1, W2, b2, Wc)
    return out[:N]


# speedup vs baseline: 3.3680x; 3.3680x over previous
import functools

import jax
import jax.numpy as jnp
from jax import lax
from jax.experimental import pallas as pl
from jax.experimental.pallas import tpu as pltpu
from jax.experimental.pallas import tpu_sc as plsc

N = 10000
NP = 10240
E = 320000
EPAD = 327680
H = 128
VOCAB = 1001

ROWS_PER_TILE_A = NP // 32
ROWS_PER_TILE_B = NP // 16
EDGES_PER_TILE = EPAD // 32
EDGE_CHUNKS = EDGES_PER_TILE // 128

_mesh = plsc.VectorSubcoreMesh(core_axis_name="c", subcore_axis_name="s")


@functools.partial(
    pl.kernel,
    mesh=_mesh,
    out_type=jax.ShapeDtypeStruct((NP, H), jnp.float32),
    scratch_types=[
        pltpu.VMEM((ROWS_PER_TILE_A,), jnp.int32),
        pltpu.VMEM((ROWS_PER_TILE_A,), jnp.int32),
        pltpu.VMEM((64, H), jnp.float32),
        pltpu.VMEM((64, H), jnp.float32),
        pltpu.SemaphoreType.DMA,
    ],
)
def _sc_embed(f0_hbm, f1_hbm, kt_hbm, vt_hbm, h_hbm, fi0, fi1, kbuf, vbuf, sem):
    c = lax.axis_index("c")
    s = lax.axis_index("s")
    wid = s * 2 + c
    n0 = wid * ROWS_PER_TILE_A
    pltpu.sync_copy(f0_hbm.at[pl.ds(n0, ROWS_PER_TILE_A)], fi0)
    pltpu.sync_copy(f1_hbm.at[pl.ds(n0, ROWS_PER_TILE_A)], fi1)
    for i in range(ROWS_PER_TILE_A // 64):
        pltpu.async_copy(kt_hbm.at[fi0.at[pl.ds(64 * i, 64)]], kbuf, sem).wait()
        pltpu.async_copy(vt_hbm.at[fi1.at[pl.ds(64 * i, 64)]], vbuf, sem).wait()

        def relu_body(r, carry):
            for j in range(H // 16):
                kbuf[r, pl.ds(16 * j, 16)] = jnp.maximum(
                    kbuf[r, pl.ds(16 * j, 16)] + vbuf[r, pl.ds(16 * j, 16)],
                    0.0)
            return carry
        lax.fori_loop(0, 64, relu_body, 0)
        pltpu.sync_copy(kbuf, h_hbm.at[pl.ds(n0 + 64 * i, 64)])


@functools.partial(
    pl.kernel,
    mesh=_mesh,
    out_type=jax.ShapeDtypeStruct((2, NP, H), jnp.float32),
    scratch_types=[
        pltpu.VMEM((4, 128), jnp.int32),
        pltpu.VMEM((4, 128), jnp.int32),
        pltpu.VMEM((128, H), jnp.float32),
        pltpu.VMEM((128, H), jnp.float32),
        pltpu.VMEM_SHARED((NP, H), jnp.float32),
        pltpu.SemaphoreType.DMA,
        pltpu.SemaphoreType.DMA,
        pltpu.SemaphoreType.DMA,
    ],
)
def _sc_edges(h_hbm, src_hbm, dst_hbm, out_hbm,
              isrc, idst, eb0, eb1, acc_sh, isem, gsem, ssem):
    c = lax.axis_index("c")
    s = lax.axis_index("s")
    r0 = s * ROWS_PER_TILE_B
    e0 = (c * 16 + s) * EDGES_PER_TILE
    eb = [eb0, eb1]

    @pl.when(c == 0)
    def _():
        pltpu.sync_copy(h_hbm.at[pl.ds(r0, ROWS_PER_TILE_B)],
                        acc_sh.at[pl.ds(r0, ROWS_PER_TILE_B)])

    @pl.when(c == 1)
    def _():
        def zero_body(r, carry):
            for j in range(H // 16):
                eb0[r, pl.ds(16 * j, 16)] = jnp.zeros((16,), jnp.float32)
            return carry
        lax.fori_loop(0, 128, zero_body, 0)
        for i in range(ROWS_PER_TILE_B // 128):
            pltpu.sync_copy(eb0, acc_sh.at[pl.ds(r0 + 128 * i, 128)])

    plsc.subcore_barrier()

    def fire_idx(j, r):
        pltpu.async_copy(src_hbm.at[pl.ds(e0 + j * 128, 128)], isrc.at[r], isem)
        pltpu.async_copy(dst_hbm.at[pl.ds(e0 + j * 128, 128)], idst.at[r], isem)

    def wait_idx(r):
        pltpu.make_async_copy(src_hbm.at[pl.ds(0, 128)], isrc.at[r], isem).wait()
        pltpu.make_async_copy(dst_hbm.at[pl.ds(0, 128)], idst.at[r], isem).wait()

    def fire_g(r, buf):
        pltpu.async_copy(h_hbm.at[isrc.at[r]], buf, gsem)

    def wait_g(buf):
        pltpu.make_async_copy(h_hbm.at[isrc.at[0]], buf, gsem).wait()

    def fire_s(r, buf):
        pltpu.async_copy(buf, acc_sh.at[idst.at[r]], ssem, add=True)

    def wait_s(buf):
        pltpu.make_async_copy(buf, acc_sh.at[idst.at[0]], ssem).wait()

    def step(j, k, fire_idx_f=True, wait_idx_f=True, wait_s_f=True,
             fire_g_f=True):
        if fire_idx_f:
            fire_idx(j + 2, (k + 2) % 4)
        if wait_idx_f:
            wait_idx((k + 1) % 4)
        if wait_s_f:
            wait_s(eb[(k + 1) % 2])
        if fire_g_f:
            fire_g((k + 1) % 4, eb[(k + 1) % 2])
        wait_g(eb[k % 2])
        fire_s(k, eb[k % 2])

    fire_idx(0, 0)
    fire_idx(1, 1)
    wait_idx(0)
    fire_g(0, eb0)
    step(0, 0, wait_s_f=False)
    step(1, 1)

    def body(t, carry):
        j = 4 * t + 2
        for k in range(4):
            step(j + k, (2 + k) % 4)
        return carry
    lax.fori_loop(0, (EDGE_CHUNKS - 8) // 4, body, 0)

    for j in range(EDGE_CHUNKS - 6, EDGE_CHUNKS):
        last = j == EDGE_CHUNKS - 1
        step(j, j % 4, fire_idx_f=j + 2 < EDGE_CHUNKS,
             wait_idx_f=not last, fire_g_f=not last)
    wait_s(eb[(EDGE_CHUNKS - 1) % 2])

    plsc.subcore_barrier()

    for i in range(ROWS_PER_TILE_B // 128):
        pltpu.sync_copy(acc_sh.at[pl.ds(r0 + 128 * i, 128)], eb0)
        pltpu.sync_copy(eb0, out_hbm.at[c].at[pl.ds(r0 + 128 * i, 128)])


BLK = 1024


def _mlp_body(ha_ref, hb_ref, w1_ref, b1_ref, w2_ref, b2_ref, wc_ref, o_ref):
    h = ha_ref[0] + hb_ref[0]
    z = jnp.dot(h, w1_ref[...], preferred_element_type=jnp.float32)
    z = jnp.maximum(z + b1_ref[...], 0.0)
    z = jnp.dot(z, w2_ref[...], preferred_element_type=jnp.float32) + b2_ref[...]
    o_ref[...] = jnp.dot(z, wc_ref[...], preferred_element_type=jnp.float32)


def _mlp(hs, W1, b1, W2, b2, Wc):
    return pl.pallas_call(
        _mlp_body,
        grid=(NP // BLK,),
        in_specs=[
            pl.BlockSpec((1, BLK, H), lambda i: (0, i, 0)),
            pl.BlockSpec((1, BLK, H), lambda i: (1, i, 0)),
            pl.BlockSpec((H, H), lambda i: (0, 0)),
            pl.BlockSpec((1, H), lambda i: (0, 0)),
            pl.BlockSpec((H, H), lambda i: (0, 0)),
            pl.BlockSpec((1, H), lambda i: (0, 0)),
            pl.BlockSpec((H, H), lambda i: (0, 0)),
        ],
        out_specs=pl.BlockSpec((BLK, H), lambda i: (i, 0)),
        out_shape=jax.ShapeDtypeStruct((NP, H), jnp.float32),
    )(hs, hs, W1, b1.reshape(1, H), W2, b2.reshape(1, H), Wc)


def kernel(feats, edge_index, key_table, val_table, W1, b1, W2, b2, Wc):
    f0 = jnp.pad(feats[:, 0], (0, NP - N))
    f1 = jnp.pad(feats[:, 1], (0, NP - N))
    pad_idx = N + (jnp.arange(EPAD - E, dtype=jnp.int32) % (NP - N))
    srcp = jnp.concatenate([edge_index[0].astype(jnp.int32), pad_idx])
    dstp = jnp.concatenate([edge_index[1].astype(jnp.int32), pad_idx])
    h = _sc_embed(f0, f1, key_table, val_table)
    hs = _sc_edges(h, srcp, dstp)
    out = _mlp(hs, W1, b1, W2, b2, Wc)
    return out[:N]


# speedup vs baseline: 3.7412x; 1.1108x over previous
import functools

import jax
import jax.numpy as jnp
from jax import lax
from jax.experimental import pallas as pl
from jax.experimental.pallas import tpu as pltpu
from jax.experimental.pallas import tpu_sc as plsc

N = 10000
NP = 10240
E = 320000
EPAD = 327680
H = 128
VOCAB = 1001

ROWS_PER_TILE_A = NP // 32
ROWS_PER_TILE_B = NP // 16
EDGES_PER_TILE = EPAD // 32
EDGE_CHUNKS = EDGES_PER_TILE // 128

_mesh = plsc.VectorSubcoreMesh(core_axis_name="c", subcore_axis_name="s")


@functools.partial(
    pl.kernel,
    mesh=_mesh,
    out_type=jax.ShapeDtypeStruct((NP, H), jnp.float32),
    scratch_types=[
        pltpu.VMEM((ROWS_PER_TILE_A,), jnp.int32),
        pltpu.VMEM((ROWS_PER_TILE_A,), jnp.int32),
        pltpu.VMEM((64, H), jnp.float32),
        pltpu.VMEM((64, H), jnp.float32),
        pltpu.VMEM((64, H), jnp.float32),
        pltpu.VMEM((64, H), jnp.float32),
        pltpu.SemaphoreType.DMA,
        pltpu.SemaphoreType.DMA,
    ],
)
def _sc_embed(f0_hbm, f1_hbm, kt_hbm, vt_hbm, h_hbm,
              fi0, fi1, kb0, kb1, vb0, vb1, gsem, wsem):
    c = lax.axis_index("c")
    s = lax.axis_index("s")
    wid = s * 2 + c
    n0 = wid * ROWS_PER_TILE_A
    kb = [kb0, kb1]
    vb = [vb0, vb1]
    nch = ROWS_PER_TILE_A // 64

    pltpu.async_copy(f0_hbm.at[pl.ds(n0, ROWS_PER_TILE_A)], fi0, gsem)
    desc = pltpu.async_copy(f1_hbm.at[pl.ds(n0, ROWS_PER_TILE_A)], fi1, gsem)
    pltpu.make_async_copy(f0_hbm.at[pl.ds(n0, ROWS_PER_TILE_A)], fi0,
                          gsem).wait()
    desc.wait()

    def fire_g(i, p):
        pltpu.async_copy(kt_hbm.at[fi0.at[pl.ds(64 * i, 64)]], kb[p], gsem)
        pltpu.async_copy(vt_hbm.at[fi1.at[pl.ds(64 * i, 64)]], vb[p], gsem)

    def wait_g(p):
        pltpu.make_async_copy(kt_hbm.at[fi0.at[pl.ds(0, 64)]], kb[p],
                              gsem).wait()
        pltpu.make_async_copy(vt_hbm.at[fi1.at[pl.ds(0, 64)]], vb[p],
                              gsem).wait()

    def wait_w(p):
        pltpu.make_async_copy(kb[p], h_hbm.at[pl.ds(n0, 64)], wsem).wait()

    fire_g(0, 0)
    for i in range(nch):
        p = i % 2
        wait_g(p)
        if i >= 1:
            wait_w(1 - p)
        if i + 1 < nch:
            fire_g(i + 1, 1 - p)

        def relu_body(r, carry):
            for j in range(H // 16):
                kb[p][r, pl.ds(16 * j, 16)] = jnp.maximum(
                    kb[p][r, pl.ds(16 * j, 16)] + vb[p][r, pl.ds(16 * j, 16)],
                    0.0)
            return carry
        lax.fori_loop(0, 64, relu_body, 0)
        pltpu.async_copy(kb[p], h_hbm.at[pl.ds(n0 + 64 * i, 64)], wsem)
    wait_w((nch - 1) % 2)


@functools.partial(
    pl.kernel,
    mesh=_mesh,
    out_type=jax.ShapeDtypeStruct((2, NP, H), jnp.float32),
    scratch_types=[
        pltpu.VMEM((4, 128), jnp.int32),
        pltpu.VMEM((4, 128), jnp.int32),
        pltpu.VMEM((128, H), jnp.float32),
        pltpu.VMEM((128, H), jnp.float32),
        pltpu.VMEM_SHARED((NP, H), jnp.float32),
        pltpu.SemaphoreType.DMA,
        pltpu.SemaphoreType.DMA,
        pltpu.SemaphoreType.DMA,
    ],
)
def _sc_edges(h_hbm, src_hbm, dst_hbm, out_hbm,
              isrc, idst, eb0, eb1, acc_sh, isem, gsem, ssem):
    c = lax.axis_index("c")
    s = lax.axis_index("s")
    r0 = s * ROWS_PER_TILE_B
    e0 = (c * 16 + s) * EDGES_PER_TILE
    eb = [eb0, eb1]

    @pl.when(c == 0)
    def _():
        pltpu.sync_copy(h_hbm.at[pl.ds(r0, ROWS_PER_TILE_B)],
                        acc_sh.at[pl.ds(r0, ROWS_PER_TILE_B)])

    @pl.when(c == 1)
    def _():
        def zero_body(r, carry):
            for j in range(H // 16):
                eb0[r, pl.ds(16 * j, 16)] = jnp.zeros((16,), jnp.float32)
            return carry
        lax.fori_loop(0, 128, zero_body, 0)
        for i in range(ROWS_PER_TILE_B // 128):
            pltpu.sync_copy(eb0, acc_sh.at[pl.ds(r0 + 128 * i, 128)])

    plsc.subcore_barrier()

    def fire_idx(j, r):
        pltpu.async_copy(src_hbm.at[pl.ds(e0 + j * 128, 128)], isrc.at[r], isem)
        pltpu.async_copy(dst_hbm.at[pl.ds(e0 + j * 128, 128)], idst.at[r], isem)

    def wait_idx(r):
        pltpu.make_async_copy(src_hbm.at[pl.ds(0, 128)], isrc.at[r], isem).wait()
        pltpu.make_async_copy(dst_hbm.at[pl.ds(0, 128)], idst.at[r], isem).wait()

    def fire_g(r, buf):
        pltpu.async_copy(h_hbm.at[isrc.at[r]], buf, gsem)

    def wait_g(buf):
        pltpu.make_async_copy(h_hbm.at[isrc.at[0]], buf, gsem).wait()

    def fire_s(r, buf):
        pltpu.async_copy(buf, acc_sh.at[idst.at[r]], ssem, add=True)

    def wait_s(buf):
        pltpu.make_async_copy(buf, acc_sh.at[idst.at[0]], ssem).wait()

    def step(j, k, fire_idx_f=True, wait_idx_f=True, wait_s_f=True,
             fire_g_f=True):
        if fire_idx_f:
            fire_idx(j + 2, (k + 2) % 4)
        if wait_idx_f:
            wait_idx((k + 1) % 4)
        if wait_s_f:
            wait_s(eb[(k + 1) % 2])
        if fire_g_f:
            fire_g((k + 1) % 4, eb[(k + 1) % 2])
        wait_g(eb[k % 2])
        fire_s(k, eb[k % 2])

    fire_idx(0, 0)
    fire_idx(1, 1)
    wait_idx(0)
    fire_g(0, eb0)
    step(0, 0, wait_s_f=False)
    step(1, 1)

    def body(t, carry):
        j = 4 * t + 2
        for k in range(4):
            step(j + k, (2 + k) % 4)
        return carry
    lax.fori_loop(0, (EDGE_CHUNKS - 8) // 4, body, 0)

    for j in range(EDGE_CHUNKS - 6, EDGE_CHUNKS):
        last = j == EDGE_CHUNKS - 1
        step(j, j % 4, fire_idx_f=j + 2 < EDGE_CHUNKS,
             wait_idx_f=not last, fire_g_f=not last)
    wait_s(eb[(EDGE_CHUNKS - 1) % 2])

    plsc.subcore_barrier()

    for i in range(ROWS_PER_TILE_B // 128):
        pltpu.sync_copy(acc_sh.at[pl.ds(r0 + 128 * i, 128)], eb0)
        pltpu.sync_copy(eb0, out_hbm.at[c].at[pl.ds(r0 + 128 * i, 128)])


BLK = 1024


def _mlp_body(ha_ref, hb_ref, w1_ref, b1_ref, w2_ref, b2_ref, wc_ref, o_ref):
    h = ha_ref[0] + hb_ref[0]
    z = jnp.dot(h, w1_ref[...], preferred_element_type=jnp.float32)
    z = jnp.maximum(z + b1_ref[...], 0.0)
    z = jnp.dot(z, w2_ref[...], preferred_element_type=jnp.float32) + b2_ref[...]
    o_ref[...] = jnp.dot(z, wc_ref[...], preferred_element_type=jnp.float32)


def _mlp(hs, W1, b1, W2, b2, Wc):
    return pl.pallas_call(
        _mlp_body,
        grid=(NP // BLK,),
        in_specs=[
            pl.BlockSpec((1, BLK, H), lambda i: (0, i, 0)),
            pl.BlockSpec((1, BLK, H), lambda i: (1, i, 0)),
            pl.BlockSpec((H, H), lambda i: (0, 0)),
            pl.BlockSpec((1, H), lambda i: (0, 0)),
            pl.BlockSpec((H, H), lambda i: (0, 0)),
            pl.BlockSpec((1, H), lambda i: (0, 0)),
            pl.BlockSpec((H, H), lambda i: (0, 0)),
        ],
        out_specs=pl.BlockSpec((BLK, H), lambda i: (i, 0)),
        out_shape=jax.ShapeDtypeStruct((NP, H), jnp.float32),
    )(hs, hs, W1, b1.reshape(1, H), W2, b2.reshape(1, H), Wc)


def kernel(feats, edge_index, key_table, val_table, W1, b1, W2, b2, Wc):
    f0 = jnp.pad(feats[:, 0], (0, NP - N))
    f1 = jnp.pad(feats[:, 1], (0, NP - N))
    pad_idx = N + (jnp.arange(EPAD - E, dtype=jnp.int32) % (NP - N))
    srcp = jnp.concatenate([edge_index[0].astype(jnp.int32), pad_idx])
    dstp = jnp.concatenate([edge_index[1].astype(jnp.int32), pad_idx])
    h = _sc_embed(f0, f1, key_table, val_table)
    hs = _sc_edges(h, srcp, dstp)
    out = _mlp(hs, W1, b1, W2, b2, Wc)
    return out[:N]


# speedup vs baseline: 3.9023x; 1.0430x over previous
import functools

import jax
import jax.numpy as jnp
from jax import lax
from jax.experimental import pallas as pl
from jax.experimental.pallas import tpu as pltpu
from jax.experimental.pallas import tpu_sc as plsc

N = 10000
NP = 10240
E = 320000
H = 128
VOCAB = 1001

ROWS_PER_TILE_A = NP // 32
ROWS_PER_TILE_B = NP // 16
EDGES_PER_TILE = 10240
EDGE_CHUNKS = EDGES_PER_TILE // 128

_mesh = plsc.VectorSubcoreMesh(core_axis_name="c", subcore_axis_name="s")


@functools.partial(
    pl.kernel,
    mesh=_mesh,
    out_type=jax.ShapeDtypeStruct((NP, H), jnp.float32),
    scratch_types=[
        pltpu.VMEM((ROWS_PER_TILE_A,), jnp.int32),
        pltpu.VMEM((ROWS_PER_TILE_A,), jnp.int32),
        pltpu.VMEM((64, H), jnp.float32),
        pltpu.VMEM((64, H), jnp.float32),
        pltpu.VMEM((64, H), jnp.float32),
        pltpu.VMEM((64, H), jnp.float32),
        pltpu.SemaphoreType.DMA,
        pltpu.SemaphoreType.DMA,
    ],
)
def _sc_embed(f0_hbm, f1_hbm, kt_hbm, vt_hbm, h_hbm,
              fi0, fi1, kb0, kb1, vb0, vb1, gsem, wsem):
    c = lax.axis_index("c")
    s = lax.axis_index("s")
    wid = s * 2 + c
    n0 = wid * ROWS_PER_TILE_A
    kb = [kb0, kb1]
    vb = [vb0, vb1]
    nch = ROWS_PER_TILE_A // 64

    pltpu.async_copy(f0_hbm.at[pl.ds(n0, ROWS_PER_TILE_A)], fi0, gsem)
    desc = pltpu.async_copy(f1_hbm.at[pl.ds(n0, ROWS_PER_TILE_A)], fi1, gsem)
    pltpu.make_async_copy(f0_hbm.at[pl.ds(n0, ROWS_PER_TILE_A)], fi0,
                          gsem).wait()
    desc.wait()

    def fire_g(i, p):
        pltpu.async_copy(kt_hbm.at[fi0.at[pl.ds(64 * i, 64)]], kb[p], gsem)
        pltpu.async_copy(vt_hbm.at[fi1.at[pl.ds(64 * i, 64)]], vb[p], gsem)

    def wait_g(p):
        pltpu.make_async_copy(kt_hbm.at[fi0.at[pl.ds(0, 64)]], kb[p],
                              gsem).wait()
        pltpu.make_async_copy(vt_hbm.at[fi1.at[pl.ds(0, 64)]], vb[p],
                              gsem).wait()

    def wait_w(p):
        pltpu.make_async_copy(kb[p], h_hbm.at[pl.ds(n0, 64)], wsem).wait()

    fire_g(0, 0)
    for i in range(nch):
        p = i % 2
        wait_g(p)
        if i >= 1:
            wait_w(1 - p)
        if i + 1 < nch:
            fire_g(i + 1, 1 - p)

        def relu_body(r, carry):
            for j in range(H // 16):
                kb[p][r, pl.ds(16 * j, 16)] = jnp.maximum(
                    kb[p][r, pl.ds(16 * j, 16)] + vb[p][r, pl.ds(16 * j, 16)],
                    0.0)
            return carry
        lax.fori_loop(0, 64, relu_body, 0)
        pltpu.async_copy(kb[p], h_hbm.at[pl.ds(n0 + 64 * i, 64)], wsem)
    wait_w((nch - 1) % 2)


@functools.partial(
    pl.kernel,
    mesh=_mesh,
    out_type=jax.ShapeDtypeStruct((2, NP, H), jnp.float32),
    scratch_types=[
        pltpu.VMEM((4, 128), jnp.int32),
        pltpu.VMEM((4, 128), jnp.int32),
        pltpu.VMEM((128, H), jnp.float32),
        pltpu.VMEM((128, H), jnp.float32),
        pltpu.VMEM_SHARED((NP, H), jnp.float32),
        pltpu.SemaphoreType.DMA,
        pltpu.SemaphoreType.DMA,
        pltpu.SemaphoreType.DMA,
    ],
)
def _sc_edges(h_hbm, ei_hbm, out_hbm,
              isrc, idst, eb0, eb1, acc_sh, isem, gsem, ssem):
    c = lax.axis_index("c")
    s = lax.axis_index("s")
    r0 = s * ROWS_PER_TILE_B
    tid = c * 16 + s
    e0 = tid * EDGES_PER_TILE
    nch = jnp.where(tid == 31, 20, EDGE_CHUNKS)
    eb = [eb0, eb1]

    @pl.when(c == 0)
    def _():
        pltpu.sync_copy(h_hbm.at[pl.ds(r0, ROWS_PER_TILE_B)],
                        acc_sh.at[pl.ds(r0, ROWS_PER_TILE_B)])

    @pl.when(c == 1)
    def _():
        def zero_body(r, carry):
            for j in range(H // 16):
                eb0[r, pl.ds(16 * j, 16)] = jnp.zeros((16,), jnp.float32)
            return carry
        lax.fori_loop(0, 128, zero_body, 0)
        for i in range(ROWS_PER_TILE_B // 128):
            pltpu.sync_copy(eb0, acc_sh.at[pl.ds(r0 + 128 * i, 128)])

    plsc.subcore_barrier()

    def fire_idx(j, r):
        pltpu.async_copy(ei_hbm.at[0].at[pl.ds(e0 + j * 128, 128)],
                         isrc.at[r], isem)
        pltpu.async_copy(ei_hbm.at[1].at[pl.ds(e0 + j * 128, 128)],
                         idst.at[r], isem)

    def wait_idx(r):
        pltpu.make_async_copy(ei_hbm.at[0].at[pl.ds(0, 128)], isrc.at[r],
                              isem).wait()
        pltpu.make_async_copy(ei_hbm.at[1].at[pl.ds(0, 128)], idst.at[r],
                              isem).wait()

    def fire_g(r, buf):
        pltpu.async_copy(h_hbm.at[isrc.at[r]], buf, gsem)

    def wait_g(buf):
        pltpu.make_async_copy(h_hbm.at[isrc.at[0]], buf, gsem).wait()

    def fire_s(r, buf):
        pltpu.async_copy(buf, acc_sh.at[idst.at[r]], ssem, add=True)

    def wait_s(buf):
        pltpu.make_async_copy(buf, acc_sh.at[idst.at[0]], ssem).wait()

    def step(j, k, fire_idx_f=True, wait_idx_f=True, wait_s_f=True,
             fire_g_f=True):
        if fire_idx_f:
            fire_idx(j + 2, (k + 2) % 4)
        if wait_idx_f:
            wait_idx((k + 1) % 4)
        if wait_s_f:
            wait_s(eb[(k + 1) % 2])
        if fire_g_f:
            fire_g((k + 1) % 4, eb[(k + 1) % 2])
        wait_g(eb[k % 2])
        fire_s(k, eb[k % 2])

    fire_idx(0, 0)
    fire_idx(1, 1)
    wait_idx(0)
    fire_g(0, eb0)
    step(0, 0, wait_s_f=False)
    step(1, 1)

    def body(t, carry):
        j = 4 * t + 2
        for k in range(4):
            step(j + k, (2 + k) % 4)
        return carry
    lax.fori_loop(0, (nch - 8) // 4, body, 0)

    for k in range(6):
        last = k == 5
        step(nch - 6 + k, (2 + k) % 4, fire_idx_f=k < 4,
             wait_idx_f=not last, fire_g_f=not last)
    wait_s(eb[(6 - 1) % 2])

    plsc.subcore_barrier()

    for i in range(ROWS_PER_TILE_B // 128):
        pltpu.sync_copy(acc_sh.at[pl.ds(r0 + 128 * i, 128)], eb0)
        pltpu.sync_copy(eb0, out_hbm.at[c].at[pl.ds(r0 + 128 * i, 128)])


BLK = 2000


def _mlp_body(ha_ref, hb_ref, w1_ref, b1_ref, w2_ref, b2_ref, wc_ref, o_ref):
    h = ha_ref[0] + hb_ref[0]
    z = jnp.dot(h, w1_ref[...], preferred_element_type=jnp.float32)
    z = jnp.maximum(z + b1_ref[...], 0.0)
    z = jnp.dot(z, w2_ref[...], preferred_element_type=jnp.float32) + b2_ref[...]
    o_ref[...] = jnp.dot(z, wc_ref[...], preferred_element_type=jnp.float32)


def _mlp(hs, W1, b1, W2, b2, Wc):
    return pl.pallas_call(
        _mlp_body,
        grid=(N // BLK,),
        in_specs=[
            pl.BlockSpec((1, BLK, H), lambda i: (0, i, 0)),
            pl.BlockSpec((1, BLK, H), lambda i: (1, i, 0)),
            pl.BlockSpec((H, H), lambda i: (0, 0)),
            pl.BlockSpec((1, H), lambda i: (0, 0)),
            pl.BlockSpec((H, H), lambda i: (0, 0)),
            pl.BlockSpec((1, H), lambda i: (0, 0)),
            pl.BlockSpec((H, H), lambda i: (0, 0)),
        ],
        out_specs=pl.BlockSpec((BLK, H), lambda i: (i, 0)),
        out_shape=jax.ShapeDtypeStruct((N, H), jnp.float32),
    )(hs, hs, W1, b1.reshape(1, H), W2, b2.reshape(1, H), Wc)


def kernel(feats, edge_index, key_table, val_table, W1, b1, W2, b2, Wc):
    f0 = jnp.pad(feats[:, 0], (0, NP - N))
    f1 = jnp.pad(feats[:, 1], (0, NP - N))
    h = _sc_embed(f0, f1, key_table, val_table)
    hs = _sc_edges(h, edge_index)
    return _mlp(hs, W1, b1, W2, b2, Wc)


# speedup vs baseline: 3.9812x; 1.0202x over previous
import functools

import jax
import jax.numpy as jnp
from jax import lax
from jax.experimental import pallas as pl
from jax.experimental.pallas import tpu as pltpu
from jax.experimental.pallas import tpu_sc as plsc

N = 10000
NP = 10240
E = 320000
H = 128
VOCAB = 1001

ROWS_PER_TILE_A = NP // 32
ROWS_PER_TILE_B = NP // 16
EDGES_PER_TILE = 10240
EDGE_CHUNKS = EDGES_PER_TILE // 128

_mesh = plsc.VectorSubcoreMesh(core_axis_name="c", subcore_axis_name="s")


@functools.partial(
    pl.kernel,
    mesh=_mesh,
    out_type=jax.ShapeDtypeStruct((NP, H), jnp.float32),
    scratch_types=[
        pltpu.VMEM((ROWS_PER_TILE_A,), jnp.int32),
        pltpu.VMEM((ROWS_PER_TILE_A,), jnp.int32),
        pltpu.VMEM((160, H), jnp.float32),
        pltpu.VMEM((160, H), jnp.float32),
        pltpu.VMEM((160, H), jnp.float32),
        pltpu.VMEM((160, H), jnp.float32),
        pltpu.SemaphoreType.DMA,
        pltpu.SemaphoreType.DMA,
    ],
)
def _sc_embed(f0_hbm, f1_hbm, kt_hbm, vt_hbm, h_hbm,
              fi0, fi1, kb0, kb1, vb0, vb1, gsem, wsem):
    c = lax.axis_index("c")
    s = lax.axis_index("s")
    wid = s * 2 + c
    n0 = wid * ROWS_PER_TILE_A
    kb = [kb0, kb1]
    vb = [vb0, vb1]
    CH = 160
    nch = ROWS_PER_TILE_A // CH

    pltpu.async_copy(f0_hbm.at[pl.ds(n0, ROWS_PER_TILE_A)], fi0, gsem)
    desc = pltpu.async_copy(f1_hbm.at[pl.ds(n0, ROWS_PER_TILE_A)], fi1, gsem)
    pltpu.make_async_copy(f0_hbm.at[pl.ds(n0, ROWS_PER_TILE_A)], fi0,
                          gsem).wait()
    desc.wait()

    def fire_g(i, p):
        pltpu.async_copy(kt_hbm.at[fi0.at[pl.ds(CH * i, CH)]], kb[p], gsem)
        pltpu.async_copy(vt_hbm.at[fi1.at[pl.ds(CH * i, CH)]], vb[p], gsem)

    def wait_g(p):
        pltpu.make_async_copy(kt_hbm.at[fi0.at[pl.ds(0, CH)]], kb[p],
                              gsem).wait()
        pltpu.make_async_copy(vt_hbm.at[fi1.at[pl.ds(0, CH)]], vb[p],
                              gsem).wait()

    def wait_w(p):
        pltpu.make_async_copy(kb[p], h_hbm.at[pl.ds(n0, CH)], wsem).wait()

    fire_g(0, 0)
    for i in range(nch):
        p = i % 2
        wait_g(p)
        if i >= 1:
            wait_w(1 - p)
        if i + 1 < nch:
            fire_g(i + 1, 1 - p)

        def relu_body(r, carry):
            for j in range(H // 16):
                kb[p][r, pl.ds(16 * j, 16)] = jnp.maximum(
                    kb[p][r, pl.ds(16 * j, 16)] + vb[p][r, pl.ds(16 * j, 16)],
                    0.0)
            return carry
        lax.fori_loop(0, CH, relu_body, 0)
        pltpu.async_copy(kb[p], h_hbm.at[pl.ds(n0 + CH * i, CH)], wsem)
    wait_w((nch - 1) % 2)


@functools.partial(
    pl.kernel,
    mesh=_mesh,
    out_type=jax.ShapeDtypeStruct((2, NP, H), jnp.float32),
    scratch_types=[
        pltpu.VMEM((4, 128), jnp.int32),
        pltpu.VMEM((4, 128), jnp.int32),
        pltpu.VMEM((128, H), jnp.float32),
        pltpu.VMEM((128, H), jnp.float32),
        pltpu.VMEM_SHARED((NP, H), jnp.float32),
        pltpu.SemaphoreType.DMA,
        pltpu.SemaphoreType.DMA,
        pltpu.SemaphoreType.DMA,
    ],
)
def _sc_edges(h_hbm, ei_hbm, out_hbm,
              isrc, idst, eb0, eb1, acc_sh, isem, gsem, ssem):
    c = lax.axis_index("c")
    s = lax.axis_index("s")
    r0 = s * ROWS_PER_TILE_B
    tid = c * 16 + s
    e0 = tid * EDGES_PER_TILE
    nch = jnp.where(tid == 31, 20, EDGE_CHUNKS)
    eb = [eb0, eb1]

    @pl.when(c == 0)
    def _():
        pltpu.sync_copy(h_hbm.at[pl.ds(r0, ROWS_PER_TILE_B)],
                        acc_sh.at[pl.ds(r0, ROWS_PER_TILE_B)])

    @pl.when(c == 1)
    def _():
        def zero_body(r, carry):
            for j in range(H // 16):
                eb0[r, pl.ds(16 * j, 16)] = jnp.zeros((16,), jnp.float32)
            return carry
        lax.fori_loop(0, 128, zero_body, 0)
        for i in range(ROWS_PER_TILE_B // 128):
            pltpu.sync_copy(eb0, acc_sh.at[pl.ds(r0 + 128 * i, 128)])

    plsc.subcore_barrier()

    def fire_idx(j, r):
        pltpu.async_copy(ei_hbm.at[0].at[pl.ds(e0 + j * 128, 128)],
                         isrc.at[r], isem)
        pltpu.async_copy(ei_hbm.at[1].at[pl.ds(e0 + j * 128, 128)],
                         idst.at[r], isem)

    def wait_idx(r):
        pltpu.make_async_copy(ei_hbm.at[0].at[pl.ds(0, 128)], isrc.at[r],
                              isem).wait()
        pltpu.make_async_copy(ei_hbm.at[1].at[pl.ds(0, 128)], idst.at[r],
                              isem).wait()

    def fire_g(r, buf):
        pltpu.async_copy(h_hbm.at[isrc.at[r]], buf, gsem)

    def wait_g(buf):
        pltpu.make_async_copy(h_hbm.at[isrc.at[0]], buf, gsem).wait()

    def fire_s(r, buf):
        pltpu.async_copy(buf, acc_sh.at[idst.at[r]], ssem, add=True)

    def wait_s(buf):
        pltpu.make_async_copy(buf, acc_sh.at[idst.at[0]], ssem).wait()

    def step(j, k, fire_idx_f=True, wait_idx_f=True, wait_s_f=True,
             fire_g_f=True):
        if fire_idx_f:
            fire_idx(j + 2, (k + 2) % 4)
        if wait_idx_f:
            wait_idx((k + 1) % 4)
        if wait_s_f:
            wait_s(eb[(k + 1) % 2])
        if fire_g_f:
            fire_g((k + 1) % 4, eb[(k + 1) % 2])
        wait_g(eb[k % 2])
        fire_s(k, eb[k % 2])

    fire_idx(0, 0)
    fire_idx(1, 1)
    wait_idx(0)
    fire_g(0, eb0)
    step(0, 0, wait_s_f=False)
    step(1, 1)

    def body(t, carry):
        j = 4 * t + 2
        for k in range(4):
            step(j + k, (2 + k) % 4)
        return carry
    lax.fori_loop(0, (nch - 8) // 4, body, 0)

    for k in range(6):
        last = k == 5
        step(nch - 6 + k, (2 + k) % 4, fire_idx_f=k < 4,
             wait_idx_f=not last, fire_g_f=not last)
    wait_s(eb[(6 - 1) % 2])

    plsc.subcore_barrier()

    for i in range(ROWS_PER_TILE_B // 128):
        pltpu.sync_copy(acc_sh.at[pl.ds(r0 + 128 * i, 128)], eb0)
        pltpu.sync_copy(eb0, out_hbm.at[c].at[pl.ds(r0 + 128 * i, 128)])


BLK = 2000


def _mlp_body(ha_ref, hb_ref, w1_ref, b1_ref, w2_ref, b2_ref, wc_ref, o_ref):
    h = ha_ref[0] + hb_ref[0]
    z = jnp.dot(h, w1_ref[...], preferred_element_type=jnp.float32)
    z = jnp.maximum(z + b1_ref[...], 0.0)
    z = jnp.dot(z, w2_ref[...], preferred_element_type=jnp.float32) + b2_ref[...]
    o_ref[...] = jnp.dot(z, wc_ref[...], preferred_element_type=jnp.float32)


def _mlp(hs, W1, b1, W2, b2, Wc):
    return pl.pallas_call(
        _mlp_body,
        grid=(N // BLK,),
        in_specs=[
            pl.BlockSpec((1, BLK, H), lambda i: (0, i, 0)),
            pl.BlockSpec((1, BLK, H), lambda i: (1, i, 0)),
            pl.BlockSpec((H, H), lambda i: (0, 0)),
            pl.BlockSpec((1, H), lambda i: (0, 0)),
            pl.BlockSpec((H, H), lambda i: (0, 0)),
            pl.BlockSpec((1, H), lambda i: (0, 0)),
            pl.BlockSpec((H, H), lambda i: (0, 0)),
        ],
        out_specs=pl.BlockSpec((BLK, H), lambda i: (i, 0)),
        out_shape=jax.ShapeDtypeStruct((N, H), jnp.float32),
    )(hs, hs, W1, b1.reshape(1, H), W2, b2.reshape(1, H), Wc)


def kernel(feats, edge_index, key_table, val_table, W1, b1, W2, b2, Wc):
    f0 = jnp.pad(feats[:, 0], (0, NP - N))
    f1 = jnp.pad(feats[:, 1], (0, NP - N))
    h = _sc_embed(f0, f1, key_table, val_table)
    hs = _sc_edges(h, edge_index)
    return _mlp(hs, W1, b1, W2, b2, Wc)
